# baseline (speedup 1.0000x reference)
import functools

import jax
import jax.numpy as jnp
from jax import lax
from jax.experimental import pallas as pl
from jax.experimental.pallas import tpu as pltpu
from jax.experimental.pallas import tpu_sc as plsc

N_QUERY = 800_000
N_MEM = 1_600_000

NC, NS, L = 2, 16, 16
NW = NC * NS

BLK = 1024
ROWS = BLK // 128
GRP = BLK // L

M_PER_T = N_MEM // NS
MFULL = M_PER_T // BLK
MTAIL = M_PER_T - MFULL * BLK

Q_PER_W = N_QUERY // NW
QFULL = Q_PER_W // BLK
QTAIL = Q_PER_W - QFULL * BLK

TBL = 589_824
TSLICE = TBL // NS
ZBLK = 4096
ZITER = TSLICE // ZBLK

SENT = 550_000


def _i32(x):
    return jnp.int32(x)


def _keys_full(src_ref, dst_ref, kidx_ref, iota2, ngroups):
    for g in range(ngroups):
        off = iota2 + _i32(g * 32)
        sv = plsc.load_gather(src_ref, [off])
        dv = plsc.load_gather(dst_ref, [off])
        kidx_ref[g // 8, pl.ds((g % 8) * 16, 16)] = sv * _i32(10) + dv


def _keys_fill(kidx_ref, g_lo, g_hi, value):
    filler = jnp.full((L,), value, jnp.int32)
    for g in range(g_lo, g_hi):
        kidx_ref[g // 8, pl.ds((g % 8) * 16, 16)] = filler


def _sc_kernel(qs, qd, ms, md, pos16, out,
               table, sbuf, dbuf, kidx, vals, qval, zbuf, pbuf, sem):
    c = lax.axis_index("c")
    s = lax.axis_index("s")
    wid = s * _i32(NC) + c
    iota2 = lax.iota(jnp.int32, L) * _i32(2)

    def zinit(i, _):
        zbuf[pl.ds(i * _i32(16), 16)] = jnp.zeros((16,), jnp.float32)
        return 0
    lax.fori_loop(_i32(0), _i32(ZBLK // 16), zinit, 0)
    for r in range(ZITER):
        pltpu.sync_copy(zbuf, table.at[pl.ds(s * _i32(TSLICE) + _i32(r * ZBLK), ZBLK)])

    pltpu.sync_copy(pos16, pbuf)
    pv = pbuf[...]
    for j in range(ROWS):
        for i in range(8):
            vals[j, pl.ds(i * 16, 16)] = pv

    plsc.subcore_barrier()

    def scat_block(b, _):
        base = pl.multiple_of((s * _i32(M_PER_T) + b * _i32(BLK)) * _i32(2), 2 * BLK)
        pltpu.sync_copy(ms.at[pl.ds(base, 2 * BLK)], sbuf)
        pltpu.sync_copy(md.at[pl.ds(base, 2 * BLK)], dbuf)
        _keys_full(sbuf, dbuf, kidx, iota2, GRP)
        copies = [pltpu.async_copy(vals.at[_i32(j)], table.at[kidx.at[_i32(j)]], sem)
                  for j in range(ROWS)]
        for cp in copies:
            cp.wait()
        return 0
    lax.fori_loop(_i32(0), _i32(MFULL), scat_block, 0)

    tbase = pl.multiple_of((s * _i32(M_PER_T) + _i32(MFULL * BLK)) * _i32(2), 8)
    pltpu.sync_copy(ms.at[pl.ds(tbase, 2 * MTAIL)], sbuf.at[pl.ds(0, 2 * MTAIL)])
    pltpu.sync_copy(md.at[pl.ds(tbase, 2 * MTAIL)], dbuf.at[pl.ds(0, 2 * MTAIL)])
    _keys_full(sbuf, dbuf, kidx, iota2, MTAIL // L)
    _keys_fill(kidx, MTAIL // L, GRP, SENT)
    copies = [pltpu.async_copy(vals.at[_i32(j)], table.at[kidx.at[_i32(j)]], sem)
              for j in range(ROWS)]
    for cp in copies:
        cp.wait()

    plsc.subcore_barrier()

    def gath_block(b, _):
        ebase = pl.multiple_of(wid * _i32(Q_PER_W) + b * _i32(BLK), 8)
        wbase = pl.multiple_of(ebase * _i32(2), 8)
        pltpu.sync_copy(qs.at[pl.ds(wbase, 2 * BLK)], sbuf)
        pltpu.sync_copy(qd.at[pl.ds(wbase, 2 * BLK)], dbuf)
        _keys_full(sbuf, dbuf, kidx, iota2, GRP)
        copies = [pltpu.async_copy(table.at[kidx.at[_i32(j)]],
                                   qval.at[pl.ds(j * 128, 128)], sem)
                  for j in range(ROWS)]
        for cp in copies:
            cp.wait()
        pltpu.sync_copy(qval, out.at[pl.ds(ebase, BLK)])
        return 0
    lax.fori_loop(_i32(0), _i32(QFULL), gath_block, 0)

    qebase = pl.multiple_of(wid * _i32(Q_PER_W) + _i32(QFULL * BLK), 8)
    qwbase = pl.multiple_of(qebase * _i32(2), 8)
    pltpu.sync_copy(qs.at[pl.ds(qwbase, 2 * QTAIL)], sbuf.at[pl.ds(0, 2 * QTAIL)])
    pltpu.sync_copy(qd.at[pl.ds(qwbase, 2 * QTAIL)], dbuf.at[pl.ds(0, 2 * QTAIL)])
    ng = QTAIL // L
    rem = QTAIL - ng * L
    _keys_full(sbuf, dbuf, kidx, iota2, ng)
    off = iota2 + _i32(ng * 32)
    sv = plsc.load_gather(sbuf, [off])
    dv = plsc.load_gather(dbuf, [off])
    lane = lax.iota(jnp.int32, L)
    kidx[ng // 8, pl.ds((ng % 8) * 16, 16)] = jnp.where(
        lane < _i32(rem), sv * _i32(10) + dv, _i32(0))
    _keys_fill(kidx, ng + 1, GRP, 0)
    copies = [pltpu.async_copy(table.at[kidx.at[_i32(j)]],
                               qval.at[pl.ds(j * 128, 128)], sem)
              for j in range(ROWS)]
    for cp in copies:
        cp.wait()
    pltpu.sync_copy(qval.at[pl.ds(0, QTAIL)], out.at[pl.ds(qebase, QTAIL)])


@functools.partial(
    pl.kernel,
    mesh=plsc.VectorSubcoreMesh(core_axis_name="c", subcore_axis_name="s",
                                num_cores=NC),
    out_type=jax.ShapeDtypeStruct((N_QUERY,), jnp.float32),
    scratch_types=[
        pltpu.VMEM_SHARED((TBL,), jnp.float32),
        pltpu.VMEM((2 * BLK,), jnp.int32),
        pltpu.VMEM((2 * BLK,), jnp.int32),
        pltpu.VMEM((ROWS, 128), jnp.int32),
        pltpu.VMEM((ROWS, 128), jnp.float32),
        pltpu.VMEM((BLK,), jnp.float32),
        pltpu.VMEM((ZBLK,), jnp.float32),
        pltpu.VMEM((16,), jnp.float32),
        pltpu.SemaphoreType.DMA,
    ],
    compiler_params=pltpu.CompilerParams(needs_layout_passes=False),
)
def _edgebank_sc(qs, qd, ms, md, pos16, out,
                 table, sbuf, dbuf, kidx, vals, qval, zbuf, pbuf, sem):
    _sc_kernel(qs, qd, ms, md, pos16, out,
               table, sbuf, dbuf, kidx, vals, qval, zbuf, pbuf, sem)


def kernel(query_edge_indices, mem_edge_index, pos_prob):
    q32 = lax.bitcast_convert_type(query_edge_indices, jnp.int32)
    m32 = lax.bitcast_convert_type(mem_edge_index, jnp.int32)
    qs = q32[0].reshape(-1)
    qd = q32[1].reshape(-1)
    ms = m32[0].reshape(-1)
    md = m32[1].reshape(-1)
    pos16 = jnp.broadcast_to(pos_prob.astype(jnp.float32), (16,))
    return _edgebank_sc(qs, qd, ms, md, pos16)


# speedup vs baseline: 6.8652x; 6.8652x over previous
import functools

import jax
import jax.numpy as jnp
from jax import lax
from jax.experimental import pallas as pl
from jax.experimental.pallas import tpu as pltpu
from jax.experimental.pallas import tpu_sc as plsc

N_QUERY = 800_000
N_MEM = 1_600_000

NC, NS, L = 2, 16, 16
NW = NC * NS

BLK = 1024
ROWS = BLK // 128
GRP = BLK // L

M_PER_T = N_MEM // NS
MFULL = M_PER_T // BLK
MTAIL = M_PER_T - MFULL * BLK

Q_PER_W = N_QUERY // NW
QFULL = Q_PER_W // BLK
QTAIL = Q_PER_W - QFULL * BLK

TBL = 589_824
TSLICE = TBL // NS
ZBLK = 4096
ZITER = TSLICE // ZBLK

SENT = 550_000


def _i32(x):
    return jnp.int32(x)


def _keys_full(src_ref, dst_ref, kidx_ref, ngroups):
    for g in range(ngroups):
        sv = src_ref[pl.ds(g * 16, 16)]
        dv = dst_ref[pl.ds(g * 16, 16)]
        kidx_ref[g // 8, pl.ds((g % 8) * 16, 16)] = sv * _i32(10) + dv


def _keys_fill(kidx_ref, g_lo, g_hi, value):
    filler = jnp.full((L,), value, jnp.int32)
    for g in range(g_lo, g_hi):
        kidx_ref[g // 8, pl.ds((g % 8) * 16, 16)] = filler


def _sc_kernel(qs, qd, ms, md, pos16, out,
               table, sbuf, dbuf, kidx, vals, qval, zbuf, pbuf, sem):
    c = lax.axis_index("c")
    s = lax.axis_index("s")
    wid = s * _i32(NC) + c

    def zinit(i, _):
        zbuf[pl.ds(i * _i32(16), 16)] = jnp.zeros((16,), jnp.float32)
        return 0
    lax.fori_loop(_i32(0), _i32(ZBLK // 16), zinit, 0)
    for r in range(ZITER):
        pltpu.sync_copy(zbuf, table.at[pl.ds(s * _i32(TSLICE) + _i32(r * ZBLK), ZBLK)])

    pltpu.sync_copy(pos16, pbuf)
    pv = pbuf[...]
    for j in range(ROWS):
        for i in range(8):
            vals[j, pl.ds(i * 16, 16)] = pv

    plsc.subcore_barrier()

    def scat_block(b, _):
        base = pl.multiple_of(s * _i32(M_PER_T) + b * _i32(BLK), 8)
        pltpu.sync_copy(ms.at[pl.ds(base, BLK)], sbuf)
        pltpu.sync_copy(md.at[pl.ds(base, BLK)], dbuf)
        _keys_full(sbuf, dbuf, kidx, GRP)
        copies = [pltpu.async_copy(vals.at[_i32(j)], table.at[kidx.at[_i32(j)]], sem)
                  for j in range(ROWS)]
        for cp in copies:
            cp.wait()
        return 0
    lax.fori_loop(_i32(0), _i32(MFULL), scat_block, 0)

    tbase = pl.multiple_of(s * _i32(M_PER_T) + _i32(MFULL * BLK), 8)
    pltpu.sync_copy(ms.at[pl.ds(tbase, MTAIL)], sbuf.at[pl.ds(0, MTAIL)])
    pltpu.sync_copy(md.at[pl.ds(tbase, MTAIL)], dbuf.at[pl.ds(0, MTAIL)])
    _keys_full(sbuf, dbuf, kidx, MTAIL // L)
    _keys_fill(kidx, MTAIL // L, GRP, SENT)
    copies = [pltpu.async_copy(vals.at[_i32(j)], table.at[kidx.at[_i32(j)]], sem)
              for j in range(ROWS)]
    for cp in copies:
        cp.wait()

    plsc.subcore_barrier()

    def gath_block(b, _):
        base = pl.multiple_of(wid * _i32(Q_PER_W) + b * _i32(BLK), 8)
        pltpu.sync_copy(qs.at[pl.ds(base, BLK)], sbuf)
        pltpu.sync_copy(qd.at[pl.ds(base, BLK)], dbuf)
        _keys_full(sbuf, dbuf, kidx, GRP)
        copies = [pltpu.async_copy(table.at[kidx.at[_i32(j)]],
                                   qval.at[pl.ds(j * 128, 128)], sem)
                  for j in range(ROWS)]
        for cp in copies:
            cp.wait()
        pltpu.sync_copy(qval, out.at[pl.ds(base, BLK)])
        return 0
    lax.fori_loop(_i32(0), _i32(QFULL), gath_block, 0)

    qbase = pl.multiple_of(wid * _i32(Q_PER_W) + _i32(QFULL * BLK), 8)
    pltpu.sync_copy(qs.at[pl.ds(qbase, QTAIL)], sbuf.at[pl.ds(0, QTAIL)])
    pltpu.sync_copy(qd.at[pl.ds(qbase, QTAIL)], dbuf.at[pl.ds(0, QTAIL)])
    ng = QTAIL // L
    rem = QTAIL - ng * L
    _keys_full(sbuf, dbuf, kidx, ng)
    sv = sbuf[pl.ds(ng * 16, 16)]
    dv = dbuf[pl.ds(ng * 16, 16)]
    lane = lax.iota(jnp.int32, L)
    kidx[ng // 8, pl.ds((ng % 8) * 16, 16)] = jnp.where(
        lane < _i32(rem), sv * _i32(10) + dv, _i32(0))
    _keys_fill(kidx, ng + 1, GRP, 0)
    copies = [pltpu.async_copy(table.at[kidx.at[_i32(j)]],
                               qval.at[pl.ds(j * 128, 128)], sem)
              for j in range(ROWS)]
    for cp in copies:
        cp.wait()
    pltpu.sync_copy(qval.at[pl.ds(0, QTAIL)], out.at[pl.ds(qbase, QTAIL)])


@functools.partial(
    pl.kernel,
    mesh=plsc.VectorSubcoreMesh(core_axis_name="c", subcore_axis_name="s",
                                num_cores=NC),
    out_type=jax.ShapeDtypeStruct((N_QUERY,), jnp.float32),
    scratch_types=[
        pltpu.VMEM_SHARED((TBL,), jnp.float32),
        pltpu.VMEM((BLK,), jnp.int32),
        pltpu.VMEM((BLK,), jnp.int32),
        pltpu.VMEM((ROWS, 128), jnp.int32),
        pltpu.VMEM((ROWS, 128), jnp.float32),
        pltpu.VMEM((BLK,), jnp.float32),
        pltpu.VMEM((ZBLK,), jnp.float32),
        pltpu.VMEM((16,), jnp.float32),
        pltpu.SemaphoreType.DMA,
    ],
)
def _edgebank_sc(qs, qd, ms, md, pos16, out,
                 table, sbuf, dbuf, kidx, vals, qval, zbuf, pbuf, sem):
    _sc_kernel(qs, qd, ms, md, pos16, out,
               table, sbuf, dbuf, kidx, vals, qval, zbuf, pbuf, sem)


def kernel(query_edge_indices, mem_edge_index, pos_prob):
    q = query_edge_indices.astype(jnp.int32)
    m = mem_edge_index.astype(jnp.int32)
    pos16 = jnp.broadcast_to(pos_prob.astype(jnp.float32), (16,))
    return _edgebank_sc(q[0], q[1], m[0], m[1], pos16)
